# baseline (speedup 1.0000x reference)
import functools

import jax
import jax.experimental.layout
import jax.numpy as jnp
from jax import lax
from jax.experimental import pallas as pl
from jax.experimental.pallas import tpu as pltpu
from jax.experimental.pallas import tpu_sc as plsc

NUM_FACTORS = 64
BATCH = 16384
NC, NS, L = 2, 16, 16
NW = NC * NS
B_PER_W = BATCH // NW
CHUNK = 256
PASSES = B_PER_W // CHUNK
GROUPS = CHUNK // L

_GATHER_DNUMS = lax.GatherDimensionNumbers(
    offset_dims=(), collapsed_slice_dims=(0,), start_index_map=(0,))


def _shuffle(x, perm):
    return lax.gather(x, perm[:, None], _GATHER_DNUMS, slice_sizes=(1,),
                      mode=lax.GatherScatterMode.PROMISE_IN_BOUNDS)


def _gmf_body(user_hbm, item_hbm, utab_hbm, itab_hbm, wb_hbm, out_hbm,
              idx_u, idx_i, u_rows, i_rows, wb_v, out_v,
              sem_u, sem_i):
    wid = lax.axis_index("s") * NC + lax.axis_index("c")
    base = wid * B_PER_W

    pltpu.sync_copy(wb_hbm, wb_v)
    pltpu.sync_copy(user_hbm.at[pl.ds(base, B_PER_W)], idx_u)
    pltpu.sync_copy(item_hbm.at[pl.ds(base, B_PER_W)], idx_i)

    w0 = wb_v[pl.ds(0, L)]
    w1 = wb_v[pl.ds(L, L)]
    w2 = wb_v[pl.ds(2 * L, L)]
    w3 = wb_v[pl.ds(3 * L, L)]
    b_vec = wb_v[pl.ds(4 * L, L)]
    lane = lax.iota(jnp.int32, L)
    perms = [lane ^ k for k in (8, 4, 2, 1)]

    for p in range(PASSES):
        off = p * CHUNK

        def fire(g, _):
            ivu = idx_u[pl.ds(off + g * L, L)]
            ivi = idx_i[pl.ds(off + g * L, L)]
            for j in range(L):
                pltpu.make_async_copy(
                    utab_hbm.at[ivu[j]], u_rows.at[g * L + j], sem_u).start()
                pltpu.make_async_copy(
                    itab_hbm.at[ivi[j]], i_rows.at[g * L + j], sem_i).start()
            return 0

        def drain(r, _):
            pltpu.make_async_copy(
                utab_hbm.at[0], u_rows.at[0], sem_u).wait()
            pltpu.make_async_copy(
                itab_hbm.at[0], i_rows.at[0], sem_i).wait()
            return 0

        lax.fori_loop(0, GROUPS, fire, 0)
        lax.fori_loop(0, CHUNK, drain, 0)

        def group(g, carry):
            acc = b_vec
            for j in range(L):
                row = g * L + j
                s = (u_rows[row, pl.ds(0, L)] * i_rows[row, pl.ds(0, L)] * w0
                     + u_rows[row, pl.ds(L, L)] * i_rows[row, pl.ds(L, L)] * w1
                     + u_rows[row, pl.ds(2 * L, L)] * i_rows[row, pl.ds(2 * L, L)] * w2
                     + u_rows[row, pl.ds(3 * L, L)] * i_rows[row, pl.ds(3 * L, L)] * w3)
                for q in perms:
                    s = s + _shuffle(s, q)
                acc = jnp.where(lane == j, s, acc)
            out_v[pl.ds(off + g * L, L)] = 1.0 / (1.0 + jnp.exp(-acc))
            return carry

        lax.fori_loop(0, GROUPS, group, 0)

    pltpu.sync_copy(out_v, out_hbm.at[pl.ds(base, B_PER_W)])


@functools.lru_cache(maxsize=1)
def _jitted():
    sharding = jax.sharding.SingleDeviceSharding(jax.devices()[0])
    fmt = lambda m2m, tiling: jax.experimental.layout.Format(
        jax.experimental.layout.Layout(major_to_minor=m2m, tiling=tiling),
        sharding)
    return jax.jit(
        _kernel_impl,
        in_shardings=(fmt((0,), ((1024,),)), fmt((0,), ((1024,),)),
                      fmt((1, 0), ((8, 128),)), fmt((1, 0), ((8, 128),)),
                      fmt((0, 1), ((1, 128),)), fmt((0,), ((128,),))))


def kernel(user, item, embed_user_mf, embed_item_mf, W, b):
    return _jitted()(user, item, embed_user_mf, embed_item_mf, W, b)


def _kernel_impl(user, item, embed_user_mf, embed_item_mf, W, b):
    wb = jnp.concatenate(
        [W.reshape(-1), jnp.broadcast_to(b.reshape(-1)[0], (L,))]).astype(jnp.float32)
    mesh = plsc.VectorSubcoreMesh(core_axis_name="c", subcore_axis_name="s")
    run = pl.kernel(
        _gmf_body,
        out_type=jax.ShapeDtypeStruct((BATCH,), jnp.float32),
        mesh=mesh,
        scratch_types=[
            pltpu.VMEM((B_PER_W,), jnp.int32),
            pltpu.VMEM((B_PER_W,), jnp.int32),
            pltpu.VMEM((CHUNK, NUM_FACTORS), jnp.float32),
            pltpu.VMEM((CHUNK, NUM_FACTORS), jnp.float32),
            pltpu.VMEM((5 * L,), jnp.float32),
            pltpu.VMEM((B_PER_W,), jnp.float32),
            pltpu.SemaphoreType.DMA,
            pltpu.SemaphoreType.DMA,
        ],
    )
    return run(user.astype(jnp.int32), item.astype(jnp.int32),
               embed_user_mf, embed_item_mf, wb)


# speedup vs baseline: 1.0037x; 1.0037x over previous
import functools

import jax
import jax.experimental.layout
import jax.numpy as jnp
from jax import lax
from jax.experimental import pallas as pl
from jax.experimental.pallas import tpu as pltpu
from jax.experimental.pallas import tpu_sc as plsc

NUM_FACTORS = 64
BATCH = 16384
NC, NS, L = 2, 16, 16
NW = NC * NS
B_PER_W = BATCH // NW
CHUNK = 256
PASSES = B_PER_W // CHUNK
GROUPS = CHUNK // L

_GATHER_DNUMS = lax.GatherDimensionNumbers(
    offset_dims=(), collapsed_slice_dims=(0,), start_index_map=(0,))


def _shuffle(x, perm):
    return lax.gather(x, perm[:, None], _GATHER_DNUMS, slice_sizes=(1,),
                      mode=lax.GatherScatterMode.PROMISE_IN_BOUNDS)


def _gmf_body(user_hbm, item_hbm, utab_hbm, itab_hbm, wb_hbm, out_hbm,
              idx_u, idx_i, u_rows, i_rows, wb_v, out_v,
              sem_u, sem_i):
    wid = lax.axis_index("s") * NC + lax.axis_index("c")
    base = wid * B_PER_W

    pltpu.sync_copy(wb_hbm, wb_v)
    pltpu.sync_copy(user_hbm.at[pl.ds(base, B_PER_W)], idx_u)
    pltpu.sync_copy(item_hbm.at[pl.ds(base, B_PER_W)], idx_i)

    w0 = wb_v[pl.ds(0, L)]
    w1 = wb_v[pl.ds(L, L)]
    w2 = wb_v[pl.ds(2 * L, L)]
    w3 = wb_v[pl.ds(3 * L, L)]
    b_vec = wb_v[pl.ds(4 * L, L)]
    lane = lax.iota(jnp.int32, L)
    perms = [lane ^ k for k in (8, 4, 2, 1)]

    for p in range(PASSES):
        off = p * CHUNK

        def fire(g, _):
            ivu = idx_u[pl.ds(off + g * L, L)]
            ivi = idx_i[pl.ds(off + g * L, L)]
            for j in range(L):
                pltpu.make_async_copy(
                    utab_hbm.at[ivu[j]], u_rows.at[g * L + j], sem_u).start()
                pltpu.make_async_copy(
                    itab_hbm.at[ivi[j]], i_rows.at[g * L + j], sem_i).start()
            return 0

        def drain(r, _):
            pltpu.make_async_copy(
                utab_hbm.at[0], u_rows.at[0], sem_u).wait()
            pltpu.make_async_copy(
                itab_hbm.at[0], i_rows.at[0], sem_i).wait()
            return 0

        lax.fori_loop(0, GROUPS, fire, 0)
        lax.fori_loop(0, CHUNK, drain, 0)

        def group(g, carry):
            acc = b_vec
            for j in range(L):
                row = g * L + j
                s = (u_rows[row, pl.ds(0, L)] * i_rows[row, pl.ds(0, L)] * w0
                     + u_rows[row, pl.ds(L, L)] * i_rows[row, pl.ds(L, L)] * w1
                     + u_rows[row, pl.ds(2 * L, L)] * i_rows[row, pl.ds(2 * L, L)] * w2
                     + u_rows[row, pl.ds(3 * L, L)] * i_rows[row, pl.ds(3 * L, L)] * w3)
                for q in perms:
                    s = s + _shuffle(s, q)
                acc = jnp.where(lane == j, s, acc)
            out_v[pl.ds(off + g * L, L)] = 1.0 / (1.0 + jnp.exp(-acc))
            return carry

        lax.fori_loop(0, GROUPS, group, 0)

    pltpu.sync_copy(out_v, out_hbm.at[pl.ds(base, B_PER_W)])


def kernel(user, item, embed_user_mf, embed_item_mf, W, b):
    fmt = jax.experimental.layout.Layout(
        major_to_minor=(1, 0), tiling=((8, 128),))
    embed_user_mf = jax.experimental.layout.with_layout_constraint(
        embed_user_mf, fmt)
    embed_item_mf = jax.experimental.layout.with_layout_constraint(
        embed_item_mf, fmt)
    wb = jnp.concatenate(
        [W.reshape(-1), jnp.broadcast_to(b.reshape(-1)[0], (L,))]).astype(jnp.float32)
    mesh = plsc.VectorSubcoreMesh(core_axis_name="c", subcore_axis_name="s")
    run = pl.kernel(
        _gmf_body,
        out_type=jax.ShapeDtypeStruct((BATCH,), jnp.float32),
        mesh=mesh,
        scratch_types=[
            pltpu.VMEM((B_PER_W,), jnp.int32),
            pltpu.VMEM((B_PER_W,), jnp.int32),
            pltpu.VMEM((CHUNK, NUM_FACTORS), jnp.float32),
            pltpu.VMEM((CHUNK, NUM_FACTORS), jnp.float32),
            pltpu.VMEM((5 * L,), jnp.float32),
            pltpu.VMEM((B_PER_W,), jnp.float32),
            pltpu.SemaphoreType.DMA,
            pltpu.SemaphoreType.DMA,
        ],
    )
    return run(user.astype(jnp.int32), item.astype(jnp.int32),
               embed_user_mf, embed_item_mf, wb)


# speedup vs baseline: 1.0088x; 1.0050x over previous
import jax
import jax.experimental.layout
import jax.numpy as jnp
from jax import lax
from jax.experimental import pallas as pl
from jax.experimental.pallas import tpu as pltpu
from jax.experimental.pallas import tpu_sc as plsc

NUM_FACTORS = 64
BATCH = 16384
NC, NS, L = 2, 16, 16
NW = NC * NS
B_PER_W = BATCH // NW
HALF = B_PER_W // 2
HGROUPS = HALF // L

_GATHER_DNUMS = lax.GatherDimensionNumbers(
    offset_dims=(), collapsed_slice_dims=(0,), start_index_map=(0,))


def _shuffle(x, perm):
    return lax.gather(x, perm[:, None], _GATHER_DNUMS, slice_sizes=(1,),
                      mode=lax.GatherScatterMode.PROMISE_IN_BOUNDS)


def _gmf_body(user_hbm, item_hbm, utab_hbm, itab_hbm, wb_hbm, out_hbm,
              idx_u, idx_i, rows, wb_v, out_v, drain_v,
              sem_u0, sem_u1, sem_i0, sem_i1):
    wid = lax.axis_index("s") * NC + lax.axis_index("c")
    base = wid * B_PER_W

    pltpu.sync_copy(wb_hbm, wb_v)
    pltpu.sync_copy(user_hbm.at[pl.ds(base, B_PER_W)], idx_u)
    pltpu.sync_copy(item_hbm.at[pl.ds(base, B_PER_W)], idx_i)

    def make_fire(sem_u, sem_i):
        def fire(g, _):
            ivu = idx_u[pl.ds(g * L, L)]
            ivi = idx_i[pl.ds(g * L, L)]
            for j in range(L):
                row = g * L + j
                pltpu.make_async_copy(
                    utab_hbm.at[ivu[j]],
                    rows.at[row, pl.ds(0, NUM_FACTORS)], sem_u).start()
                pltpu.make_async_copy(
                    itab_hbm.at[ivi[j]],
                    rows.at[row, pl.ds(NUM_FACTORS, NUM_FACTORS)], sem_i).start()
            return 0
        return fire

    lax.fori_loop(0, HGROUPS, make_fire(sem_u0, sem_i0), 0)
    lax.fori_loop(HGROUPS, 2 * HGROUPS, make_fire(sem_u1, sem_i1), 0)

    w0 = wb_v[pl.ds(0, L)]
    w1 = wb_v[pl.ds(L, L)]
    w2 = wb_v[pl.ds(2 * L, L)]
    w3 = wb_v[pl.ds(3 * L, L)]
    b_vec = wb_v[pl.ds(4 * L, L)]
    lane = lax.iota(jnp.int32, L)
    perms = [lane ^ k for k in (8, 4, 2, 1)]
    F = NUM_FACTORS

    def group(g, carry):
        acc = b_vec
        for j in range(L):
            row = g * L + j
            s = (rows[row, pl.ds(0, L)] * rows[row, pl.ds(F, L)] * w0
                 + rows[row, pl.ds(L, L)] * rows[row, pl.ds(F + L, L)] * w1
                 + rows[row, pl.ds(2 * L, L)] * rows[row, pl.ds(F + 2 * L, L)] * w2
                 + rows[row, pl.ds(3 * L, L)] * rows[row, pl.ds(F + 3 * L, L)] * w3)
            for q in perms:
                s = s + _shuffle(s, q)
            acc = jnp.where(lane == j, s, acc)
        out_v[pl.ds(g * L, L)] = 1.0 / (1.0 + jnp.exp(-acc))
        return carry

    pltpu.make_async_copy(out_hbm, drain_v, sem_u0).wait()
    pltpu.make_async_copy(out_hbm, drain_v, sem_i0).wait()
    lax.fori_loop(0, HGROUPS, group, 0)

    pltpu.make_async_copy(out_hbm, drain_v, sem_u1).wait()
    pltpu.make_async_copy(out_hbm, drain_v, sem_i1).wait()
    lax.fori_loop(HGROUPS, 2 * HGROUPS, group, 0)

    pltpu.sync_copy(out_v, out_hbm.at[pl.ds(base, B_PER_W)])


def kernel(user, item, embed_user_mf, embed_item_mf, W, b):
    fmt = jax.experimental.layout.Layout(
        major_to_minor=(1, 0), tiling=((8, 128),))
    embed_user_mf = jax.experimental.layout.with_layout_constraint(
        embed_user_mf, fmt)
    embed_item_mf = jax.experimental.layout.with_layout_constraint(
        embed_item_mf, fmt)
    wb = jnp.concatenate(
        [W.reshape(-1), jnp.broadcast_to(b.reshape(-1)[0], (L,))]).astype(jnp.float32)
    mesh = plsc.VectorSubcoreMesh(core_axis_name="c", subcore_axis_name="s")
    run = pl.kernel(
        _gmf_body,
        out_type=jax.ShapeDtypeStruct((BATCH,), jnp.float32),
        mesh=mesh,
        scratch_types=[
            pltpu.VMEM((B_PER_W,), jnp.int32),
            pltpu.VMEM((B_PER_W,), jnp.int32),
            pltpu.VMEM((B_PER_W, 2 * NUM_FACTORS), jnp.float32),
            pltpu.VMEM((5 * L,), jnp.float32),
            pltpu.VMEM((B_PER_W,), jnp.float32),
            pltpu.VMEM((HALF * NUM_FACTORS,), jnp.float32),
            pltpu.SemaphoreType.DMA,
            pltpu.SemaphoreType.DMA,
            pltpu.SemaphoreType.DMA,
            pltpu.SemaphoreType.DMA,
        ],
    )
    return run(user.astype(jnp.int32), item.astype(jnp.int32),
               embed_user_mf, embed_item_mf, wb)


# speedup vs baseline: 1.0099x; 1.0011x over previous
import jax
import jax.experimental.layout
import jax.numpy as jnp
from jax import lax
from jax.experimental import pallas as pl
from jax.experimental.pallas import tpu as pltpu
from jax.experimental.pallas import tpu_sc as plsc

NUM_FACTORS = 64
BATCH = 16384
NC, NS, L = 2, 16, 16
NW = NC * NS
B_PER_W = BATCH // NW
HALF = B_PER_W // 2
HGROUPS = HALF // L

_GATHER_DNUMS = lax.GatherDimensionNumbers(
    offset_dims=(), collapsed_slice_dims=(0,), start_index_map=(0,))


def _shuffle(x, perm):
    return lax.gather(x, perm[:, None], _GATHER_DNUMS, slice_sizes=(1,),
                      mode=lax.GatherScatterMode.PROMISE_IN_BOUNDS)


def _gmf_body(user_hbm, item_hbm, utab_hbm, itab_hbm, wb_hbm, out_hbm,
              idx_u, idx_i, rows, wb_v, out_v, drain_v,
              sem_u0, sem_u1, sem_i0, sem_i1):
    wid = lax.axis_index("s") * NC + lax.axis_index("c")
    base = wid * B_PER_W

    pltpu.sync_copy(wb_hbm, wb_v)
    pltpu.sync_copy(user_hbm.at[pl.ds(base, B_PER_W)], idx_u)
    pltpu.sync_copy(item_hbm.at[pl.ds(base, B_PER_W)], idx_i)

    def make_fire(sem_u, sem_i):
        def fire(g, _):
            ivu = idx_u[pl.ds(g * L, L)]
            ivi = idx_i[pl.ds(g * L, L)]
            H = NUM_FACTORS // 2
            for j in range(L):
                row = g * L + j
                pltpu.make_async_copy(
                    utab_hbm.at[ivu[j], pl.ds(0, H)],
                    rows.at[row, pl.ds(0, H)], sem_u).start()
                pltpu.make_async_copy(
                    utab_hbm.at[ivu[j], pl.ds(H, H)],
                    rows.at[row, pl.ds(H, H)], sem_u).start()
                pltpu.make_async_copy(
                    itab_hbm.at[ivi[j], pl.ds(0, H)],
                    rows.at[row, pl.ds(NUM_FACTORS, H)], sem_i).start()
                pltpu.make_async_copy(
                    itab_hbm.at[ivi[j], pl.ds(H, H)],
                    rows.at[row, pl.ds(NUM_FACTORS + H, H)], sem_i).start()
            return 0
        return fire

    lax.fori_loop(0, HGROUPS, make_fire(sem_u0, sem_i0), 0)
    lax.fori_loop(HGROUPS, 2 * HGROUPS, make_fire(sem_u1, sem_i1), 0)

    w0 = wb_v[pl.ds(0, L)]
    w1 = wb_v[pl.ds(L, L)]
    w2 = wb_v[pl.ds(2 * L, L)]
    w3 = wb_v[pl.ds(3 * L, L)]
    b_vec = wb_v[pl.ds(4 * L, L)]
    lane = lax.iota(jnp.int32, L)
    perms = [lane ^ k for k in (8, 4, 2, 1)]
    F = NUM_FACTORS

    def group(g, carry):
        acc = b_vec
        for j in range(L):
            row = g * L + j
            s = (rows[row, pl.ds(0, L)] * rows[row, pl.ds(F, L)] * w0
                 + rows[row, pl.ds(L, L)] * rows[row, pl.ds(F + L, L)] * w1
                 + rows[row, pl.ds(2 * L, L)] * rows[row, pl.ds(F + 2 * L, L)] * w2
                 + rows[row, pl.ds(3 * L, L)] * rows[row, pl.ds(F + 3 * L, L)] * w3)
            for q in perms:
                s = s + _shuffle(s, q)
            acc = jnp.where(lane == j, s, acc)
        out_v[pl.ds(g * L, L)] = 1.0 / (1.0 + jnp.exp(-acc))
        return carry

    pltpu.make_async_copy(out_hbm, drain_v, sem_u0).wait()
    pltpu.make_async_copy(out_hbm, drain_v, sem_i0).wait()
    lax.fori_loop(0, HGROUPS, group, 0)

    pltpu.make_async_copy(out_hbm, drain_v, sem_u1).wait()
    pltpu.make_async_copy(out_hbm, drain_v, sem_i1).wait()
    lax.fori_loop(HGROUPS, 2 * HGROUPS, group, 0)

    pltpu.sync_copy(out_v, out_hbm.at[pl.ds(base, B_PER_W)])


def kernel(user, item, embed_user_mf, embed_item_mf, W, b):
    fmt = jax.experimental.layout.Layout(
        major_to_minor=(1, 0), tiling=((8, 128),))
    embed_user_mf = jax.experimental.layout.with_layout_constraint(
        embed_user_mf, fmt)
    embed_item_mf = jax.experimental.layout.with_layout_constraint(
        embed_item_mf, fmt)
    wb = jnp.concatenate(
        [W.reshape(-1), jnp.broadcast_to(b.reshape(-1)[0], (L,))]).astype(jnp.float32)
    mesh = plsc.VectorSubcoreMesh(core_axis_name="c", subcore_axis_name="s")
    run = pl.kernel(
        _gmf_body,
        out_type=jax.ShapeDtypeStruct((BATCH,), jnp.float32),
        mesh=mesh,
        scratch_types=[
            pltpu.VMEM((B_PER_W,), jnp.int32),
            pltpu.VMEM((B_PER_W,), jnp.int32),
            pltpu.VMEM((B_PER_W, 2 * NUM_FACTORS), jnp.float32),
            pltpu.VMEM((5 * L,), jnp.float32),
            pltpu.VMEM((B_PER_W,), jnp.float32),
            pltpu.VMEM((HALF * NUM_FACTORS,), jnp.float32),
            pltpu.SemaphoreType.DMA,
            pltpu.SemaphoreType.DMA,
            pltpu.SemaphoreType.DMA,
            pltpu.SemaphoreType.DMA,
        ],
    )
    return run(user.astype(jnp.int32), item.astype(jnp.int32),
               embed_user_mf, embed_item_mf, wb)
